# baseline (speedup 1.0000x reference)
import functools

import jax
import jax.numpy as jnp
from jax import lax
from jax.experimental import pallas as pl
from jax.experimental.pallas import tpu as pltpu
from jax.experimental.pallas import tpu_sc as plsc

BATCH = 16384
EMBED_DIM = 32
NUM_CORES = 2
NUM_SUBCORES = 16
NUM_WORKERS = NUM_CORES * NUM_SUBCORES
BPW = BATCH // NUM_WORKERS
CHUNK = 128
NCHUNK = BPW // CHUNK
LANES = 16
NGROUP = BPW // LANES


def _mf_body(uidx_hbm, iidx_hbm, uemb_hbm, iemb_hbm, ulin_hbm, ilin_hbm,
             out_hbm, uidx_v, iidx_v, urows_v, irows_v, ubias_v, ibias_v,
             out_v, sem):
  wid = lax.axis_index("s") * NUM_CORES + lax.axis_index("c")

  pltpu.sync_copy(uidx_hbm.at[wid], uidx_v)
  pltpu.sync_copy(iidx_hbm.at[wid], iidx_v)

  copies = []
  for c in range(NCHUNK):
    rows = pl.ds(c * CHUNK, CHUNK)
    copies.append(pltpu.async_copy(uemb_hbm.at[uidx_v.at[c]],
                                   urows_v.at[rows, :], sem))
    copies.append(pltpu.async_copy(iemb_hbm.at[iidx_v.at[c]],
                                   irows_v.at[rows, :], sem))
    copies.append(pltpu.async_copy(ulin_hbm.at[uidx_v.at[c]],
                                   ubias_v.at[rows], sem))
    copies.append(pltpu.async_copy(ilin_hbm.at[iidx_v.at[c]],
                                   ibias_v.at[rows], sem))
  for cp in copies:
    cp.wait()

  iota16 = lax.iota(jnp.int32, 16)

  def group(g, carry):
    rvec = g * LANES + iota16
    acc = jnp.zeros((LANES,), jnp.float32)
    for d in range(EMBED_DIM):
      cvec = jnp.full((LANES,), d, jnp.int32)
      u = plsc.load_gather(urows_v, [rvec, cvec])
      it = plsc.load_gather(irows_v, [rvec, cvec])
      acc = acc + u * it
    bu = ubias_v[pl.ds(g * LANES, LANES)]
    bi = ibias_v[pl.ds(g * LANES, LANES)]
    x = acc + bu + bi
    out_v[pl.ds(g * LANES, LANES)] = 1.0 / (1.0 + jnp.exp(-x))
    return carry

  lax.fori_loop(0, NGROUP, group, 0)

  pltpu.sync_copy(out_v, out_hbm.at[pl.ds(wid * BPW, BPW)])


@jax.jit
def _mf_call(uidx, iidx, user_embed, item_embed, user_lin, item_lin):
  mesh = plsc.VectorSubcoreMesh(core_axis_name="c", subcore_axis_name="s")
  fn = pl.kernel(
      _mf_body,
      out_type=jax.ShapeDtypeStruct((BATCH,), jnp.float32),
      mesh=mesh,
      scratch_types=[
          pltpu.VMEM((NCHUNK, CHUNK), jnp.int32),
          pltpu.VMEM((NCHUNK, CHUNK), jnp.int32),
          pltpu.VMEM((BPW, EMBED_DIM), jnp.float32),
          pltpu.VMEM((BPW, EMBED_DIM), jnp.float32),
          pltpu.VMEM((BPW,), jnp.float32),
          pltpu.VMEM((BPW,), jnp.float32),
          pltpu.VMEM((BPW,), jnp.float32),
          pltpu.SemaphoreType.DMA,
      ],
      compiler_params=pltpu.CompilerParams(needs_layout_passes=False,
                                           use_tc_tiling_on_sc=False),
  )
  return fn(uidx, iidx, user_embed, item_embed, user_lin, item_lin)


def kernel(user_tensor, item_tensor, user_embed, item_embed, user_lin,
           item_lin):
  uidx = user_tensor.astype(jnp.int32).reshape(NUM_WORKERS, NCHUNK, CHUNK)
  iidx = item_tensor.astype(jnp.int32).reshape(NUM_WORKERS, NCHUNK, CHUNK)
  return _mf_call(uidx, iidx, user_embed, item_embed,
                  user_lin.reshape(-1), item_lin.reshape(-1))
